# baseline (speedup 1.0000x reference)
import functools

import jax
import jax.numpy as jnp
from jax import lax
from jax.experimental import pallas as pl
from jax.experimental.pallas import tpu as pltpu
from jax.experimental.pallas import tpu_sc as plsc

B = 16384
L = 200
D = 32
NC = 2
NS = 16
NW = NC * NS
RPW = B // NW
C = 8
NCHUNK = RPW // C
NUM_ROWS = 1000000

DT_CHUNK = 768
DT_NFULL = NUM_ROWS // DT_CHUNK
DT_TAIL = NUM_ROWS - DT_NFULL * DT_CHUNK
DT_OUT = DT_CHUNK * D
DT_PAD = D + 1


def _detile_body(t32_hbm, tail_hbm, out_hbm, in0, in1, stage_v, out_v,
                 sem0, sem1):
    wid = lax.axis_index("s") * NC + lax.axis_index("c")

    in_bufs = (in0, in1)
    sems = (sem0, sem1)

    def in2d(b):
        return in_bufs[b]

    iota = lax.iota(jnp.int32, 16)

    nloc = (DT_NFULL + NW - 1) // NW

    def start_load(i, b):
        g = wid + i * NW

        @pl.when(g < DT_NFULL)
        def _():
            l0 = g * DT_CHUNK
            for gg in range(4):
                pltpu.async_copy(
                    t32_hbm.at[pl.ds(8 * gg, 8), pl.ds(l0, DT_CHUNK)],
                    in2d(b).at[pl.ds(8 * gg, 8), pl.ds(0, DT_CHUNK)],
                    sems[b])

    def wait_load(i, b):
        g = wid + i * NW

        @pl.when(g < DT_NFULL)
        def _():
            for gg in range(4):
                pltpu.make_async_copy(
                    t32_hbm.at[pl.ds(0, 8), pl.ds(0, DT_CHUNK)],
                    in2d(b).at[pl.ds(0, 8), pl.ds(0, DT_CHUNK)],
                    sems[b]).wait()

    def compute(i, b):
        g = wid + i * NW

        @pl.when(g < DT_NFULL)
        def _():
            in_v = in_bufs[b]

            base = iota * DT_PAD
            for d in range(32):
                idx_init = base + d

                def do_blk(lb, idx, d=d):
                    x = in_v[d, pl.ds(lb * 16, 16)]
                    plsc.store_scatter(stage_v, [idx], x)
                    return idx + 16 * DT_PAD

                lax.fori_loop(0, DT_CHUNK // 16, do_blk, idx_init,
                              unroll=8)

            def repack(q, carry2):
                v0 = stage_v[pl.ds(DT_PAD * q, 16)]
                v1 = stage_v[pl.ds(DT_PAD * q + 16, 16)]
                out_v[pl.ds(D * q, 16)] = v0
                out_v[pl.ds(D * q + 16, 16)] = v1
                return carry2

            lax.fori_loop(0, DT_CHUNK, repack, 0, unroll=8)
            pltpu.sync_copy(out_v, out_hbm.at[pl.ds(g * DT_OUT, DT_OUT)])

    start_load(0, 0)

    def loop(i, carry):
        b = lax.rem(i, 2)
        for bb in range(2):
            @pl.when(b == bb)
            def _():
                wait_load(i, bb)
                start_load(i + 1, 1 - bb)
                compute(i, bb)
        return carry

    lax.fori_loop(0, nloc, loop, 0)

    @pl.when(wid == NW - 1)
    def _():
        pltpu.sync_copy(tail_hbm, out_v.at[pl.ds(0, DT_TAIL * D)])
        pltpu.sync_copy(out_v.at[pl.ds(0, DT_TAIL * D)],
                        out_hbm.at[pl.ds(DT_NFULL * DT_OUT, DT_TAIL * D)])


def _pool_body(data_hbm, table_hbm, out_hbm,
               idx0, idx1, rows0, rows1, out_v, sem0, sem1):
    wid = lax.axis_index("s") * NC + lax.axis_index("c")
    base = wid * RPW

    idx_bufs = (idx0, idx1)
    rows_bufs = (rows0, rows1)
    sems = (sem0, sem1)

    def start_gather(g, b):
        pltpu.sync_copy(data_hbm.at[pl.ds(base + g * C, C)], idx_bufs[b])
        for c in range(C):
            pltpu.async_copy(table_hbm.at[idx_bufs[b].at[c]],
                             rows_bufs[b].at[c], sems[b])

    def wait_gather(b):
        for c in range(C):
            pltpu.make_async_copy(table_hbm.at[idx_bufs[b].at[c]],
                                  rows_bufs[b].at[c], sems[b]).wait()

    def accumulate(rows, g):
        for c in range(C):
            def body(j, accs, c=c):
                a0, a1, b0, b1 = accs
                r = 2 * j
                a0 = a0 + rows[c, r, pl.ds(0, 16)]
                a1 = a1 + rows[c, r, pl.ds(16, 16)]
                b0 = b0 + rows[c, r + 1, pl.ds(0, 16)]
                b1 = b1 + rows[c, r + 1, pl.ds(16, 16)]
                return (a0, a1, b0, b1)

            z = jnp.zeros((16,), jnp.float32)
            a0, a1, b0, b1 = lax.fori_loop(0, L // 2, body, (z, z, z, z),
                                           unroll=4)
            out_v[c, pl.ds(0, 16)] = a0 + b0
            out_v[c, pl.ds(16, 16)] = a1 + b1
        pltpu.sync_copy(out_v, out_hbm.at[pl.ds(base + g * C, C)])

    start_gather(0, 0)
    start_gather(1, 1)

    def outer(g2, carry):
        for b in range(2):
            g = 2 * g2 + b
            wait_gather(b)
            accumulate(rows_bufs[b], g)
            start_gather(g + 2, b)
        return carry

    lax.fori_loop(0, NCHUNK // 2 - 1, outer, 0)

    for b in range(2):
        g = NCHUNK - 2 + b
        wait_gather(b)
        accumulate(rows_bufs[b], g)


@functools.partial(jax.jit)
def kernel(data, lengths, table):
    del lengths
    mesh = plsc.VectorSubcoreMesh(core_axis_name="c", subcore_axis_name="s")

    t32 = table.T
    tail = table[DT_NFULL * DT_CHUNK:].reshape(-1)

    detile = pl.kernel(
        _detile_body,
        out_type=jax.ShapeDtypeStruct((NUM_ROWS * D,), jnp.float32),
        mesh=mesh,
        compiler_params=pltpu.CompilerParams(use_tc_tiling_on_sc=True,
                                             needs_layout_passes=False),
        scratch_types=[
            pltpu.VMEM((32, DT_CHUNK), jnp.float32),
            pltpu.VMEM((32, DT_CHUNK), jnp.float32),
            pltpu.VMEM((DT_CHUNK * DT_PAD,), jnp.float32),
            pltpu.VMEM((DT_OUT,), jnp.float32),
            pltpu.SemaphoreType.DMA,
            pltpu.SemaphoreType.DMA,
        ],
    )
    table_lin = detile(t32, tail).reshape(NUM_ROWS, D)

    run = pl.kernel(
        _pool_body,
        out_type=jax.ShapeDtypeStruct((B, D), jnp.float32),
        mesh=mesh,
        compiler_params=pltpu.CompilerParams(use_tc_tiling_on_sc=False),
        scratch_types=[
            pltpu.VMEM((C, L), jnp.int32),
            pltpu.VMEM((C, L), jnp.int32),
            pltpu.VMEM((C, L, D), jnp.float32),
            pltpu.VMEM((C, L, D), jnp.float32),
            pltpu.VMEM((C, D), jnp.float32),
            pltpu.SemaphoreType.DMA,
            pltpu.SemaphoreType.DMA,
        ],
    )
    return run(data, table_lin)


# speedup vs baseline: 1.6247x; 1.6247x over previous
import functools

import jax
import jax.numpy as jnp
from jax import lax
from jax.experimental import pallas as pl
from jax.experimental.pallas import tpu as pltpu
from jax.experimental.pallas import tpu_sc as plsc

B = 16384
L = 200
D = 32
NC = 2
NS = 16
NW = NC * NS
RPW = B // NW
C = 8
NCHUNK = RPW // C
NUM_ROWS = 1000000

DT_CHUNK = 768
DT_NFULL = NUM_ROWS // DT_CHUNK
DT_TAIL = NUM_ROWS - DT_NFULL * DT_CHUNK
DT_OUT = DT_CHUNK * D
DT_PAD = D + 1


def _detile_body(t32_hbm, tail_hbm, out_hbm, in0, in1, stage_v, out_v,
                 sem0, sem1):
    wid = lax.axis_index("s") * NC + lax.axis_index("c")

    in_bufs = (in0, in1)
    sems = (sem0, sem1)

    def in2d(b):
        return in_bufs[b]

    iota = lax.iota(jnp.int32, 16)

    nloc = (DT_NFULL + NW - 1) // NW

    def start_load(i, b):
        g = wid + i * NW

        @pl.when(g < DT_NFULL)
        def _():
            l0 = g * DT_CHUNK
            for gg in range(4):
                pltpu.async_copy(
                    t32_hbm.at[pl.ds(8 * gg, 8), pl.ds(l0, DT_CHUNK)],
                    in2d(b).at[pl.ds(8 * gg, 8), pl.ds(0, DT_CHUNK)],
                    sems[b])

    def wait_load(i, b):
        g = wid + i * NW

        @pl.when(g < DT_NFULL)
        def _():
            for gg in range(4):
                pltpu.make_async_copy(
                    t32_hbm.at[pl.ds(0, 8), pl.ds(0, DT_CHUNK)],
                    in2d(b).at[pl.ds(0, 8), pl.ds(0, DT_CHUNK)],
                    sems[b]).wait()

    def compute(i, b):
        g = wid + i * NW

        @pl.when(g < DT_NFULL)
        def _():
            in_v = in_bufs[b]

            base = iota * DT_PAD
            for d in range(32):
                @functools.partial(plsc.parallel_loop, 0, DT_CHUNK // 16,
                                   unroll=4, carry=base + d)
                def _(lb, idx, d=d):
                    x = in_v[d, pl.ds(lb * 16, 16)]
                    plsc.store_scatter(stage_v, [idx], x)
                    return idx + 16 * DT_PAD

            def repack(q, carry2):
                v0 = stage_v[pl.ds(DT_PAD * q, 16)]
                v1 = stage_v[pl.ds(DT_PAD * q + 16, 16)]
                out_v[pl.ds(D * q, 16)] = v0
                out_v[pl.ds(D * q + 16, 16)] = v1
                return carry2

            lax.fori_loop(0, DT_CHUNK, repack, 0, unroll=4)
            pltpu.sync_copy(out_v, out_hbm.at[pl.ds(g * DT_OUT, DT_OUT)])

    start_load(0, 0)

    def loop(i, carry):
        b = lax.rem(i, 2)
        for bb in range(2):
            @pl.when(b == bb)
            def _():
                wait_load(i, bb)
                start_load(i + 1, 1 - bb)
                compute(i, bb)
        return carry

    lax.fori_loop(0, nloc, loop, 0)

    @pl.when(wid == NW - 1)
    def _():
        pltpu.sync_copy(tail_hbm, out_v.at[pl.ds(0, DT_TAIL * D)])
        pltpu.sync_copy(out_v.at[pl.ds(0, DT_TAIL * D)],
                        out_hbm.at[pl.ds(DT_NFULL * DT_OUT, DT_TAIL * D)])


def _pool_body(data_hbm, table_hbm, out_hbm,
               idx0, idx1, rows0, rows1, out_v, sem0, sem1):
    wid = lax.axis_index("s") * NC + lax.axis_index("c")
    base = wid * RPW

    idx_bufs = (idx0, idx1)
    rows_bufs = (rows0, rows1)
    sems = (sem0, sem1)

    def start_gather(g, b):
        pltpu.sync_copy(data_hbm.at[pl.ds(base + g * C, C)], idx_bufs[b])
        for c in range(C):
            pltpu.async_copy(table_hbm.at[idx_bufs[b].at[c]],
                             rows_bufs[b].at[c], sems[b])

    def wait_gather(b):
        for c in range(C):
            pltpu.make_async_copy(table_hbm.at[idx_bufs[b].at[c]],
                                  rows_bufs[b].at[c], sems[b]).wait()

    def accumulate(rows, g):
        for c in range(C):
            def body(j, accs, c=c):
                a0, a1, b0, b1 = accs
                r = 2 * j
                a0 = a0 + rows[c, r, pl.ds(0, 16)]
                a1 = a1 + rows[c, r, pl.ds(16, 16)]
                b0 = b0 + rows[c, r + 1, pl.ds(0, 16)]
                b1 = b1 + rows[c, r + 1, pl.ds(16, 16)]
                return (a0, a1, b0, b1)

            z = jnp.zeros((16,), jnp.float32)
            a0, a1, b0, b1 = lax.fori_loop(0, L // 2, body, (z, z, z, z),
                                           unroll=4)
            out_v[c, pl.ds(0, 16)] = a0 + b0
            out_v[c, pl.ds(16, 16)] = a1 + b1
        pltpu.sync_copy(out_v, out_hbm.at[pl.ds(base + g * C, C)])

    start_gather(0, 0)
    start_gather(1, 1)

    def outer(g2, carry):
        for b in range(2):
            g = 2 * g2 + b
            wait_gather(b)
            accumulate(rows_bufs[b], g)
            start_gather(g + 2, b)
        return carry

    lax.fori_loop(0, NCHUNK // 2 - 1, outer, 0)

    for b in range(2):
        g = NCHUNK - 2 + b
        wait_gather(b)
        accumulate(rows_bufs[b], g)


@functools.partial(jax.jit)
def kernel(data, lengths, table):
    del lengths
    mesh = plsc.VectorSubcoreMesh(core_axis_name="c", subcore_axis_name="s")

    t32 = table.T
    tail = table[DT_NFULL * DT_CHUNK:].reshape(-1)

    detile = pl.kernel(
        _detile_body,
        out_type=jax.ShapeDtypeStruct((NUM_ROWS * D,), jnp.float32),
        mesh=mesh,
        compiler_params=pltpu.CompilerParams(use_tc_tiling_on_sc=True,
                                             needs_layout_passes=False),
        scratch_types=[
            pltpu.VMEM((32, DT_CHUNK), jnp.float32),
            pltpu.VMEM((32, DT_CHUNK), jnp.float32),
            pltpu.VMEM((DT_CHUNK * DT_PAD,), jnp.float32),
            pltpu.VMEM((DT_OUT,), jnp.float32),
            pltpu.SemaphoreType.DMA,
            pltpu.SemaphoreType.DMA,
        ],
    )
    table_lin = detile(t32, tail).reshape(NUM_ROWS, D)

    run = pl.kernel(
        _pool_body,
        out_type=jax.ShapeDtypeStruct((B, D), jnp.float32),
        mesh=mesh,
        compiler_params=pltpu.CompilerParams(use_tc_tiling_on_sc=False),
        scratch_types=[
            pltpu.VMEM((C, L), jnp.int32),
            pltpu.VMEM((C, L), jnp.int32),
            pltpu.VMEM((C, L, D), jnp.float32),
            pltpu.VMEM((C, L, D), jnp.float32),
            pltpu.VMEM((C, D), jnp.float32),
            pltpu.SemaphoreType.DMA,
            pltpu.SemaphoreType.DMA,
        ],
    )
    return run(data, table_lin)


# speedup vs baseline: 2.0015x; 1.2319x over previous
import functools

import jax
import jax.numpy as jnp
from jax import lax
from jax.experimental import pallas as pl
from jax.experimental.pallas import tpu as pltpu
from jax.experimental.pallas import tpu_sc as plsc

B = 16384
L = 200
D = 32
NC = 2
NS = 16
NW = NC * NS
RPW = B // NW
C = 8
NCHUNK = RPW // C
NUM_ROWS = 1000000

DT_CHUNK = 768
DT_NFULL = NUM_ROWS // DT_CHUNK
DT_TAIL = NUM_ROWS - DT_NFULL * DT_CHUNK
DT_OUT = DT_CHUNK * D
DT_PAD = D + 1


def _detile_body(t32_hbm, tail_hbm, out_hbm, in0, in1, stage_v, out_v,
                 sem0, sem1):
    wid = lax.axis_index("s") * NC + lax.axis_index("c")

    in_bufs = (in0, in1)
    sems = (sem0, sem1)

    def in2d(b):
        return in_bufs[b]

    iota = lax.iota(jnp.int32, 16)

    nloc = (DT_NFULL + NW - 1) // NW

    def start_load(i, b):
        g = wid + i * NW

        @pl.when(g < DT_NFULL)
        def _():
            l0 = g * DT_CHUNK
            for gg in range(4):
                pltpu.async_copy(
                    t32_hbm.at[pl.ds(8 * gg, 8), pl.ds(l0, DT_CHUNK)],
                    in2d(b).at[pl.ds(8 * gg, 8), pl.ds(0, DT_CHUNK)],
                    sems[b])

    def wait_load(i, b):
        g = wid + i * NW

        @pl.when(g < DT_NFULL)
        def _():
            for gg in range(4):
                pltpu.make_async_copy(
                    t32_hbm.at[pl.ds(0, 8), pl.ds(0, DT_CHUNK)],
                    in2d(b).at[pl.ds(0, 8), pl.ds(0, DT_CHUNK)],
                    sems[b]).wait()

    def compute(i, b):
        g = wid + i * NW

        @pl.when(g < DT_NFULL)
        def _():
            in_v = in_bufs[b]

            base = iota * DT_PAD
            for d in range(32):
                @functools.partial(plsc.parallel_loop, 0, DT_CHUNK // 16,
                                   unroll=4, carry=base + d)
                def _(lb, idx, d=d):
                    x = in_v[d, pl.ds(lb * 16, 16)]
                    plsc.store_scatter(stage_v, [idx], x)
                    return idx + 16 * DT_PAD

            @functools.partial(plsc.parallel_loop, 0, DT_CHUNK,
                               unroll=4)
            def _(q):
                v0 = stage_v[pl.ds(DT_PAD * q, 16)]
                v1 = stage_v[pl.ds(DT_PAD * q + 16, 16)]
                out_v[pl.ds(D * q, 16)] = v0
                out_v[pl.ds(D * q + 16, 16)] = v1
            pltpu.sync_copy(out_v, out_hbm.at[pl.ds(g * DT_OUT, DT_OUT)])

    start_load(0, 0)

    def loop(i, carry):
        b = lax.rem(i, 2)
        for bb in range(2):
            @pl.when(b == bb)
            def _():
                wait_load(i, bb)
                start_load(i + 1, 1 - bb)
                compute(i, bb)
        return carry

    lax.fori_loop(0, nloc, loop, 0)

    @pl.when(wid == NW - 1)
    def _():
        pltpu.sync_copy(tail_hbm, out_v.at[pl.ds(0, DT_TAIL * D)])
        pltpu.sync_copy(out_v.at[pl.ds(0, DT_TAIL * D)],
                        out_hbm.at[pl.ds(DT_NFULL * DT_OUT, DT_TAIL * D)])


def _pool_body(data_hbm, table_hbm, out_hbm,
               idx0, idx1, rows0, rows1, out_v, sem0, sem1):
    wid = lax.axis_index("s") * NC + lax.axis_index("c")
    base = wid * RPW

    idx_bufs = (idx0, idx1)
    rows_bufs = (rows0, rows1)
    sems = (sem0, sem1)

    def start_gather(g, b):
        pltpu.sync_copy(data_hbm.at[pl.ds(base + g * C, C)], idx_bufs[b])
        for c in range(C):
            pltpu.async_copy(table_hbm.at[idx_bufs[b].at[c]],
                             rows_bufs[b].at[c], sems[b])

    def wait_gather(b):
        for c in range(C):
            pltpu.make_async_copy(table_hbm.at[idx_bufs[b].at[c]],
                                  rows_bufs[b].at[c], sems[b]).wait()

    def accumulate(rows, g):
        for c in range(C):
            def body(j, accs, c=c):
                a0, a1, b0, b1 = accs
                r = 2 * j
                a0 = a0 + rows[c, r, pl.ds(0, 16)]
                a1 = a1 + rows[c, r, pl.ds(16, 16)]
                b0 = b0 + rows[c, r + 1, pl.ds(0, 16)]
                b1 = b1 + rows[c, r + 1, pl.ds(16, 16)]
                return (a0, a1, b0, b1)

            z = jnp.zeros((16,), jnp.float32)
            a0, a1, b0, b1 = lax.fori_loop(0, L // 2, body, (z, z, z, z),
                                           unroll=4)
            out_v[c, pl.ds(0, 16)] = a0 + b0
            out_v[c, pl.ds(16, 16)] = a1 + b1
        pltpu.sync_copy(out_v, out_hbm.at[pl.ds(base + g * C, C)])

    start_gather(0, 0)
    start_gather(1, 1)

    def outer(g2, carry):
        for b in range(2):
            g = 2 * g2 + b
            wait_gather(b)
            accumulate(rows_bufs[b], g)
            start_gather(g + 2, b)
        return carry

    lax.fori_loop(0, NCHUNK // 2 - 1, outer, 0)

    for b in range(2):
        g = NCHUNK - 2 + b
        wait_gather(b)
        accumulate(rows_bufs[b], g)


@functools.partial(jax.jit)
def kernel(data, lengths, table):
    del lengths
    mesh = plsc.VectorSubcoreMesh(core_axis_name="c", subcore_axis_name="s")

    t32 = table.T
    tail = table[DT_NFULL * DT_CHUNK:].reshape(-1)

    detile = pl.kernel(
        _detile_body,
        out_type=jax.ShapeDtypeStruct((NUM_ROWS * D,), jnp.float32),
        mesh=mesh,
        compiler_params=pltpu.CompilerParams(use_tc_tiling_on_sc=True,
                                             needs_layout_passes=False),
        scratch_types=[
            pltpu.VMEM((32, DT_CHUNK), jnp.float32),
            pltpu.VMEM((32, DT_CHUNK), jnp.float32),
            pltpu.VMEM((DT_CHUNK * DT_PAD,), jnp.float32),
            pltpu.VMEM((DT_OUT,), jnp.float32),
            pltpu.SemaphoreType.DMA,
            pltpu.SemaphoreType.DMA,
        ],
    )
    table_lin = detile(t32, tail).reshape(NUM_ROWS, D)

    run = pl.kernel(
        _pool_body,
        out_type=jax.ShapeDtypeStruct((B, D), jnp.float32),
        mesh=mesh,
        compiler_params=pltpu.CompilerParams(use_tc_tiling_on_sc=False),
        scratch_types=[
            pltpu.VMEM((C, L), jnp.int32),
            pltpu.VMEM((C, L), jnp.int32),
            pltpu.VMEM((C, L, D), jnp.float32),
            pltpu.VMEM((C, L, D), jnp.float32),
            pltpu.VMEM((C, D), jnp.float32),
            pltpu.SemaphoreType.DMA,
            pltpu.SemaphoreType.DMA,
        ],
    )
    return run(data, table_lin)
